# baseline (speedup 1.0000x reference)
import jax
import jax.numpy as jnp
from jax import lax
from jax.experimental import pallas as pl
from jax.experimental.pallas import tpu as pltpu
from jax.experimental.pallas import tpu_sc as plsc

N = 10000
E = 160000
D = 256
DH = 128
N2 = 10240
NC = 2
NS = 16
NW = NC * NS
CH = 128
NCHUNK = E // CH
NCHP = 1280
CPW = NCHP // NW
ROWS_PER_TILE = N2 // NS

_mesh = plsc.VectorSubcoreMesh(
    core_axis_name="c", subcore_axis_name="s", num_cores=NC, num_subcores=NS
)
_sc_params = pltpu.CompilerParams(needs_layout_passes=False)


def _deg_body(src_hbm, dst_hbm, out_hbm, srcall, dstall, dego, degi):
    c = lax.axis_index("c")
    s = lax.axis_index("s")
    wid = s * NC + c
    zeros16 = jnp.zeros((16,), jnp.float32)
    ones16 = jnp.ones((16,), jnp.float32)

    pltpu.sync_copy(src_hbm.at[pl.ds(wid * CPW, CPW)], srcall)
    pltpu.sync_copy(dst_hbm.at[pl.ds(wid * CPW, CPW)], dstall)

    def zero_body(i, _):
        dego[pl.ds(i * 16, 16)] = zeros16
        degi[pl.ds(i * 16, 16)] = zeros16
        return 0

    lax.fori_loop(0, N2 // 16, zero_body, 0)

    def chunk_body(i, _):
        for j in range(CH // 16):
            si = srcall[i, pl.ds(16 * j, 16)]
            plsc.addupdate_scatter(dego, [si], ones16)
            di = dstall[i, pl.ds(16 * j, 16)]
            plsc.addupdate_scatter(degi, [di], ones16)
        return 0

    lax.fori_loop(0, CPW, chunk_body, 0)
    pltpu.sync_copy(dego, out_hbm.at[c, s, 0])
    pltpu.sync_copy(degi, out_hbm.at[c, s, 1])


_deg = pl.kernel(
    _deg_body,
    out_type=jax.ShapeDtypeStruct((NC, NS, 2, N2), jnp.float32),
    mesh=_mesh,
    scratch_types=[
        pltpu.VMEM((CPW, CH), jnp.int32),
        pltpu.VMEM((CPW, CH), jnp.int32),
        pltpu.VMEM((N2,), jnp.float32),
        pltpu.VMEM((N2,), jnp.float32),
    ],
    compiler_params=_sc_params,
)


def _agg_body(hs_hbm, src_hbm, dst_hbm, zrows_hbm, out_hbm,
              srcb0, dstb0, rows0, agg_sh, g0):
    c = lax.axis_index("c")
    s = lax.axis_index("s")
    off = c * N2

    pltpu.sync_copy(zrows_hbm, agg_sh.at[pl.ds(s * ROWS_PER_TILE,
                                               ROWS_PER_TILE)])
    plsc.subcore_barrier()

    def adjust(buf):
        for j in range(CH // 16):
            sl = pl.ds(16 * j, 16)
            buf[sl] = buf[sl] + off

    nch = NCHUNK // NS + jnp.where(s < NCHUNK - (NCHUNK // NS) * NS, 1, 0)
    cbase = s * (NCHUNK // NS) + jnp.minimum(s, NCHUNK - (NCHUNK // NS) * NS)

    def chunk_body(i, _):
        ch = cbase + i
        pltpu.sync_copy(src_hbm.at[ch], srcb0)
        pltpu.sync_copy(dst_hbm.at[ch], dstb0)
        adjust(srcb0)
        pltpu.async_copy(hs_hbm.at[srcb0], rows0, g0).wait()
        pltpu.sync_copy(rows0, agg_sh.at[dstb0], add=True)
        return 0

    lax.fori_loop(0, nch, chunk_body, 0)
    plsc.subcore_barrier()
    pltpu.sync_copy(
        agg_sh.at[pl.ds(s * ROWS_PER_TILE, ROWS_PER_TILE)],
        out_hbm.at[pl.ds(c * N2 + s * ROWS_PER_TILE, ROWS_PER_TILE)],
    )


_agg = pl.kernel(
    _agg_body,
    out_type=jax.ShapeDtypeStruct((NC * N2, DH), jnp.float32),
    mesh=_mesh,
    scratch_types=[
        pltpu.VMEM((CH,), jnp.int32),
        pltpu.VMEM((CH,), jnp.int32),
        pltpu.VMEM((CH, DH), jnp.float32),
        pltpu.VMEM_SHARED((N2, DH), jnp.float32),
        pltpu.SemaphoreType.DMA,
    ],
    compiler_params=_sc_params,
)


_HI = jax.lax.Precision.HIGHEST
_BN = 1024
_NB = N2 // _BN
_NORM_BN = 256


def _mm1_body(degp_ref, x_ref, w_ref, out_ref, ns_ref, nd_ref, ns_sc, nd_sc):
    b = pl.program_id(0)
    p = pl.program_id(1)

    @pl.when((b == 0) & (p == 0))
    def _():
        ii = lax.broadcasted_iota(jnp.int32, (_NORM_BN, _NORM_BN), 0)
        jj = lax.broadcasted_iota(jnp.int32, (_NORM_BN, _NORM_BN), 1)
        ident = jnp.where(ii == jj, 1.0, 0.0)

        def nbody(i, _):
            sl = pl.ds(i * _NORM_BN, _NORM_BN)
            d = jnp.sum(degp_ref[:, :, sl], axis=0)
            cols = lax.dot_general(ident, d, (((1,), (1,)), ((), ())),
                                   precision=_HI)
            do = cols[:, 0:1]
            di = cols[:, 1:2]
            ns_sc[sl, :] = jnp.where(
                do > 0.0, lax.rsqrt(jnp.maximum(do, 1e-12)), 0.0)
            nd_sc[sl, :] = jnp.where(
                di > 0.0, lax.rsqrt(jnp.maximum(di, 1e-12)), 0.0)
            return 0

        lax.fori_loop(0, N2 // _NORM_BN, nbody, 0)

    nsb = ns_sc[pl.ds(b * _BN, _BN), :]
    h = jnp.dot(x_ref[...], w_ref[...], precision=_HI)
    hs = h * nsb
    out_ref[...] = jnp.where(p == 0, hs[:, :DH], hs[:, DH:])
    ns_ref[...] = nsb
    nd_ref[...] = nd_sc[pl.ds(b * _BN, _BN), :]


def _mm1(degp, x, W1):
    return pl.pallas_call(
        _mm1_body,
        grid=(_NB, NC),
        in_specs=[
            pl.BlockSpec((NW, 2, N2), lambda b, p: (0, 0, 0)),
            pl.BlockSpec((_BN, D), lambda b, p: (b, 0)),
            pl.BlockSpec((D, D), lambda b, p: (0, 0)),
        ],
        out_specs=[
            pl.BlockSpec((_BN, DH), lambda b, p: (p * _NB + b, 0)),
            pl.BlockSpec((_BN, 1), lambda b, p: (b, 0)),
            pl.BlockSpec((_BN, 1), lambda b, p: (b, 0)),
        ],
        out_shape=[
            jax.ShapeDtypeStruct((NC * N2, DH), jnp.float32),
            jax.ShapeDtypeStruct((N2, 1), jnp.float32),
            jax.ShapeDtypeStruct((N2, 1), jnp.float32),
        ],
        scratch_shapes=[
            pltpu.VMEM((N2, 1), jnp.float32),
            pltpu.VMEM((N2, 1), jnp.float32),
        ],
    )(degp, x, W1)


def _mm2_body(a0_ref, a1_ref, nd_ref, b1_ref, w_ref, ns_ref, out_ref):
    p = pl.program_id(1)
    a = jnp.concatenate([a0_ref[...], a1_ref[...]], axis=1)
    t = jnp.maximum(a * nd_ref[...] + b1_ref[...], 0.0)
    h = jnp.dot(t, w_ref[...], precision=_HI)
    hs = h * ns_ref[...]
    out_ref[...] = jnp.where(p == 0, hs[:, :DH], hs[:, DH:])


def _mm2(agg1, nd, b1, W2, ns):
    return pl.pallas_call(
        _mm2_body,
        grid=(_NB, NC),
        in_specs=[
            pl.BlockSpec((_BN, DH), lambda b, p: (b, 0)),
            pl.BlockSpec((_BN, DH), lambda b, p: (b + _NB, 0)),
            pl.BlockSpec((_BN, 1), lambda b, p: (b, 0)),
            pl.BlockSpec((1, D), lambda b, p: (0, 0)),
            pl.BlockSpec((D, D), lambda b, p: (0, 0)),
            pl.BlockSpec((_BN, 1), lambda b, p: (b, 0)),
        ],
        out_specs=pl.BlockSpec((_BN, DH), lambda b, p: (p * _NB + b, 0)),
        out_shape=jax.ShapeDtypeStruct((NC * N2, DH), jnp.float32),
    )(agg1, agg1, nd, b1, W2, ns)


def _final_body(a0_ref, a1_ref, nd_ref, b2_ref, out_ref):
    a = jnp.concatenate([a0_ref[...], a1_ref[...]], axis=1)
    out_ref[...] = a * nd_ref[...] + b2_ref[...]


def _final(agg2, nd, b2):
    return pl.pallas_call(
        _final_body,
        grid=(_NB,),
        in_specs=[
            pl.BlockSpec((_BN, DH), lambda b: (b, 0)),
            pl.BlockSpec((_BN, DH), lambda b: (b + _NB, 0)),
            pl.BlockSpec((_BN, 1), lambda b: (b, 0)),
            pl.BlockSpec((1, D), lambda b: (0, 0)),
        ],
        out_specs=pl.BlockSpec((_BN, D), lambda b: (b, 0)),
        out_shape=jax.ShapeDtypeStruct((N, D), jnp.float32),
    )(agg2, agg2, nd, b2)


@jax.jit
def kernel(x, edge_index, W1, b1, W2, b2):
    pad = jnp.full((NCHP * CH - E,), N, jnp.int32)
    srcd = jnp.concatenate([edge_index[0], pad]).reshape(NCHP, CH)
    dstd = jnp.concatenate([edge_index[1], pad]).reshape(NCHP, CH)

    degp = _deg(srcd, dstd).reshape(NC * NS, 2, N2)
    zrows = jnp.zeros((ROWS_PER_TILE, DH), jnp.float32)

    hs1, ns, nd = _mm1(degp, x, W1)
    agg1 = _agg(hs1, srcd, dstd, zrows)

    hs2 = _mm2(agg1, nd, b1.reshape(1, D), W2, ns)
    agg2 = _agg(hs2, srcd, dstd, zrows)

    return _final(agg2, nd, b2.reshape(1, D))
